# baseline (speedup 1.0000x reference)
import functools

import jax
import jax.numpy as jnp
from jax import lax
from jax.experimental import pallas as pl
from jax.experimental.pallas import tpu as pltpu
from jax.experimental.pallas import tpu_sc as plsc

N_NODES = 10000
N_HEDGES = 10000
N_GRAPHS = 64
C = 128
NPAD = 10112
TRASH = 10000
NC = 2
NS = 16
NW = NC * NS
CHUNK = 64
PART = 16
ROWS_PER_TILE = NPAD // NS
FRAC0 = 0.65
F32 = jnp.float32


def _chunk_split(nnz):
    total = -(-nnz // CHUNK)
    per_tile = -(-total // NS)
    nch0 = max(PART, int(round(per_tile * FRAC0 / PART)) * PART)
    nch1 = -(-(total - NS * nch0) // NS)
    nch1 = max(PART, -(-nch1 // PART) * PART)
    return nch0, nch1


def _zero_rows(ref, nrows):
    z = jnp.zeros((16,), F32)

    def bi(i, carry):
        for j in range(C // 16):
            ref[i, pl.ds(j * 16, 16)] = z
        return carry

    lax.fori_loop(0, nrows, bi, 0)


def _fill_1d(ref, nvec, value):
    v = jnp.full((16,), value, F32)

    def bj(j, c2):
        ref[pl.ds(j * 16, 16)] = v
        return c2

    lax.fori_loop(0, nvec, bj, 0)


@functools.lru_cache(maxsize=None)
def _make_spmm(t_rows, nch0, nch1, with_counts):
    mesh = plsc.VectorSubcoreMesh(core_axis_name="c", subcore_axis_name="s")

    out_type = [jax.ShapeDtypeStruct((NC, NPAD, C), F32)]
    scratch = [
        pltpu.VMEM_SHARED((NPAD, C), F32),
        pltpu.VMEM((PART, CHUNK), jnp.int32),
        pltpu.VMEM((PART, CHUNK), jnp.int32),
        pltpu.VMEM((CHUNK, C), F32),
        pltpu.VMEM((CHUNK, C), F32),
        pltpu.SemaphoreType.DMA,
        pltpu.SemaphoreType.DMA,
    ]
    if with_counts:
        out_type += [jax.ShapeDtypeStruct((NC * NPAD,), F32),
                     jax.ShapeDtypeStruct((NC * NPAD,), F32)]
        scratch += [
            pltpu.VMEM_SHARED((NPAD,), F32),
            pltpu.VMEM_SHARED((NPAD,), F32),
            pltpu.VMEM((CHUNK,), F32),
            pltpu.VMEM((640,), F32),
        ]

    def prologue(acc, rows0, s):
        base = s * ROWS_PER_TILE
        _zero_rows(rows0, CHUNK)
        for k in range(ROWS_PER_TILE // CHUNK):
            pltpu.sync_copy(rows0, acc.at[pl.ds(base + k * CHUNK, CHUNK)])
        rem = ROWS_PER_TILE % CHUNK
        if rem:
            pltpu.sync_copy(rows0.at[pl.ds(0, rem)],
                            acc.at[pl.ds(base + ROWS_PER_TILE - rem, rem)])
        return base

    def make_part_loop(T, isrc, idst, acc, isv, idv, rows0, rows1, sem0, sem1,
                       extra):
        def gather(j, buf, sem):
            pltpu.async_copy(T.at[isv.at[j]], buf, sem)

        def gwait(j, buf, sem):
            pltpu.make_async_copy(T.at[isv.at[j]], buf, sem).wait()

        def scatter(j, buf):
            pltpu.sync_copy(buf, acc.at[idv.at[j]], add=True)
            if extra is not None:
                dh, bh, ones_v = extra
                pltpu.sync_copy(ones_v, dh.at[isv.at[j]], add=True)
                pltpu.sync_copy(ones_v, bh.at[idv.at[j]], add=True)

        def pair_step(i, carry):
            j0 = 2 * i
            gather(j0 + 1, rows1, sem1)
            gwait(j0, rows0, sem0)
            scatter(j0, rows0)

            @pl.when(i < PART // 2 - 1)
            def _next():
                gather(j0 + 2, rows0, sem0)

            gwait(j0 + 1, rows1, sem1)
            scatter(j0 + 1, rows1)
            return carry

        def run(start, nch):
            for p in range(nch // PART):
                pltpu.sync_copy(isrc.at[pl.ds(start + p * PART, PART)], isv)
                pltpu.sync_copy(idst.at[pl.ds(start + p * PART, PART)], idv)
                gather(0, rows0, sem0)
                lax.fori_loop(0, PART // 2, pair_step, 0)

        return run

    def core_dispatch(run, c, s):
        @pl.when(c == 0)
        def _c0():
            run(s * nch0, nch0)

        @pl.when(c == 1)
        def _c1():
            run(NS * nch0 + s * nch1, nch1)

    if with_counts:
        def body(T, isrc, idst, P, Dh, Bh,
                 acc, isv, idv, rows0, rows1, sem0, sem1,
                 dh, bh, ones_v, zline):
            c = lax.axis_index("c")
            s = lax.axis_index("s")
            base = prologue(acc, rows0, s)
            _fill_1d(ones_v, CHUNK // 16, 1.0)
            _fill_1d(zline, 40, 0.0)
            pltpu.sync_copy(zline.at[pl.ds(0, ROWS_PER_TILE)],
                            dh.at[pl.ds(base, ROWS_PER_TILE)])
            pltpu.sync_copy(zline.at[pl.ds(0, ROWS_PER_TILE)],
                            bh.at[pl.ds(base, ROWS_PER_TILE)])
            plsc.subcore_barrier()
            run = make_part_loop(T, isrc, idst, acc, isv, idv, rows0, rows1,
                                 sem0, sem1, (dh, bh, ones_v))
            core_dispatch(run, c, s)
            plsc.subcore_barrier()
            pltpu.sync_copy(acc.at[pl.ds(base, ROWS_PER_TILE)],
                            P.at[c, pl.ds(base, ROWS_PER_TILE)])

            @pl.when(s == 0)
            def _drain_hists():
                pltpu.sync_copy(dh, Dh.at[pl.ds(c * NPAD, NPAD)])
                pltpu.sync_copy(bh, Bh.at[pl.ds(c * NPAD, NPAD)])
    else:
        def body(T, isrc, idst, P,
                 acc, isv, idv, rows0, rows1, sem0, sem1):
            c = lax.axis_index("c")
            s = lax.axis_index("s")
            base = prologue(acc, rows0, s)
            plsc.subcore_barrier()
            run = make_part_loop(T, isrc, idst, acc, isv, idv, rows0, rows1,
                                 sem0, sem1, None)
            core_dispatch(run, c, s)
            plsc.subcore_barrier()
            pltpu.sync_copy(acc.at[pl.ds(base, ROWS_PER_TILE)],
                            P.at[c, pl.ds(base, ROWS_PER_TILE)])

    return pl.kernel(body, out_type=tuple(out_type), mesh=mesh,
                     scratch_types=tuple(scratch))


def _tc_matmul_body(x_ref, w_ref, o_ref):
    o_ref[...] = jnp.dot(x_ref[...], w_ref[...], preferred_element_type=F32)


def _tc_inv_body(npad_e, dh_ref, bh_ref, dinv_ref, binv_ref):
    row = lax.broadcasted_iota(jnp.int32, (NPAD // C, C), 0)
    col = lax.broadcasted_iota(jnp.int32, (NPAD // C, C), 1)
    corr = jnp.where((row == 0) & (col == 0), float(npad_e), 0.0)
    d = dh_ref[0] + dh_ref[1] - corr
    b = bh_ref[0] + bh_ref[1]
    dinv_ref[...] = jnp.where(d > 0, 1.0 / d, 0.0)
    binv_ref[...] = jnp.where(b > 0, 1.0 / b, 0.0)


def _tc_scale_body(p_ref, inv_ref, o_ref):
    o_ref[...] = (p_ref[0] + p_ref[1]) * inv_ref[...]


def _tc_layer_end_body(p_ref, inv_ref, b_ref, w_ref, o_ref):
    h = jax.nn.relu((p_ref[0] + p_ref[1]) * inv_ref[...] + b_ref[...])
    o_ref[...] = jnp.dot(h, w_ref[...], preferred_element_type=F32)


def _tc_final_body(p_ref, inv_ref, b_ref, batch_ref, wfc_ref, bfc_ref, o_ref):
    h = jax.nn.relu((p_ref[0] + p_ref[1]) * inv_ref[...] + b_ref[...])
    gids = lax.broadcasted_iota(jnp.int32, (N_GRAPHS, 1), 0)
    oht = (gids == batch_ref[...]).astype(F32)
    sums = jnp.dot(oht, h, preferred_element_type=F32,
                   precision=lax.Precision.HIGHEST)
    cnts = jnp.dot(oht, jnp.ones((NPAD, 1), F32),
                   preferred_element_type=F32, precision=lax.Precision.HIGHEST)
    pooled = sums / jnp.maximum(cnts, 1.0)
    o_ref[...] = jnp.dot(pooled, wfc_ref[...],
                         preferred_element_type=F32) + bfc_ref[...]


def _tc(body, out_shape, *args):
    return pl.pallas_call(body, out_shape=out_shape)(*args)


def kernel(x, edge_index, batch, W1, b1, W2, b2, Wfc, bfc):
    nnz = edge_index.shape[1]
    nch0, nch1 = _chunk_split(nnz)
    total_chunks = NS * (nch0 + nch1)
    nnz_pad = total_chunks * CHUNK
    npad_e = nnz_pad - nnz

    node = edge_index[0]
    edge = edge_index[1]
    pad0 = jnp.zeros((npad_e,), jnp.int32)
    padT = jnp.full((npad_e,), TRASH, jnp.int32)
    shape2 = (total_chunks, CHUNK)
    node0 = jnp.concatenate([node, pad0]).reshape(shape2)
    nodeT = jnp.concatenate([node, padT]).reshape(shape2)
    edge0 = jnp.concatenate([edge, pad0]).reshape(shape2)
    edgeT = jnp.concatenate([edge, padT]).reshape(shape2)

    spmm_c = _make_spmm(N_NODES, nch0, nch1, True)
    spmm_n = _make_spmm(NPAD, nch0, nch1, False)

    t1 = _tc(_tc_matmul_body, jax.ShapeDtypeStruct((N_NODES, C), F32), x, W1)
    P, Dh, Bh = spmm_c(t1, node0, edgeT)
    dinv, binv = _tc(
        functools.partial(_tc_inv_body, npad_e),
        (jax.ShapeDtypeStruct((NPAD // C, C), F32),) * 2,
        Dh.reshape(NC, NPAD // C, C), Bh.reshape(NC, NPAD // C, C))
    dinv_col = dinv.reshape(NPAD, 1)
    binv_col = binv.reshape(NPAD, 1)
    m1 = _tc(_tc_scale_body, jax.ShapeDtypeStruct((NPAD, C), F32), P, binv_col)
    (P,) = spmm_n(m1, edge0, nodeT)
    t2 = _tc(_tc_layer_end_body, jax.ShapeDtypeStruct((NPAD, C), F32),
             P, dinv_col, b1.reshape(1, C), W2)

    (P,) = spmm_n(t2, node0, edgeT)
    m2 = _tc(_tc_scale_body, jax.ShapeDtypeStruct((NPAD, C), F32), P, binv_col)
    (P,) = spmm_n(m2, edge0, nodeT)

    batch_row = jnp.concatenate(
        [batch, jnp.full((NPAD - N_NODES,), N_GRAPHS, jnp.int32)]).reshape(1, NPAD)
    out = _tc(_tc_final_body, jax.ShapeDtypeStruct((N_GRAPHS, 1), F32),
              P, dinv_col, b2.reshape(1, C), batch_row, Wfc, bfc.reshape(1, 1))
    return out.reshape(-1)


# speedup vs baseline: 1.0312x; 1.0312x over previous
import functools

import jax
import jax.numpy as jnp
from jax import lax
from jax.experimental import pallas as pl
from jax.experimental.pallas import tpu as pltpu
from jax.experimental.pallas import tpu_sc as plsc

N_NODES = 10000
N_HEDGES = 10000
N_GRAPHS = 64
C = 128
NPAD = 10112
TRASH = 10000
NC = 2
NS = 16
NW = NC * NS
CHUNK = 64
PART = 16
ROWS_PER_TILE = NPAD // NS
FRAC0 = 0.8
F32 = jnp.float32


def _chunk_split(nnz):
    total = -(-nnz // CHUNK)
    per_tile = -(-total // NS)
    nch0 = max(PART, int(round(per_tile * FRAC0 / PART)) * PART)
    nch1 = -(-(total - NS * nch0) // NS)
    nch1 = max(PART, -(-nch1 // PART) * PART)
    return nch0, nch1


def _zero_rows(ref, nrows):
    z = jnp.zeros((16,), F32)

    def bi(i, carry):
        for j in range(C // 16):
            ref[i, pl.ds(j * 16, 16)] = z
        return carry

    lax.fori_loop(0, nrows, bi, 0)


def _fill_1d(ref, nvec, value):
    v = jnp.full((16,), value, F32)

    def bj(j, c2):
        ref[pl.ds(j * 16, 16)] = v
        return c2

    lax.fori_loop(0, nvec, bj, 0)


@functools.lru_cache(maxsize=None)
def _make_spmm(t_rows, nch0, nch1, with_counts):
    mesh = plsc.VectorSubcoreMesh(core_axis_name="c", subcore_axis_name="s")

    out_type = [jax.ShapeDtypeStruct((NC, NPAD, C), F32)]
    scratch = [
        pltpu.VMEM_SHARED((NPAD, C), F32),
        pltpu.VMEM((PART, CHUNK), jnp.int32),
        pltpu.VMEM((PART, CHUNK), jnp.int32),
        pltpu.VMEM((CHUNK, C), F32),
        pltpu.VMEM((CHUNK, C), F32),
        pltpu.VMEM((CHUNK, C), F32),
        pltpu.VMEM((CHUNK, C), F32),
        pltpu.SemaphoreType.DMA,
        pltpu.SemaphoreType.DMA,
        pltpu.SemaphoreType.DMA,
        pltpu.SemaphoreType.DMA,
    ]
    if with_counts:
        out_type += [jax.ShapeDtypeStruct((NC * NPAD,), F32),
                     jax.ShapeDtypeStruct((NC * NPAD,), F32)]
        scratch += [
            pltpu.VMEM_SHARED((NPAD,), F32),
            pltpu.VMEM_SHARED((NPAD,), F32),
            pltpu.VMEM((CHUNK,), F32),
            pltpu.VMEM((640,), F32),
        ]

    def prologue(acc, rows0, s):
        base = s * ROWS_PER_TILE
        _zero_rows(rows0, CHUNK)
        for k in range(ROWS_PER_TILE // CHUNK):
            pltpu.sync_copy(rows0, acc.at[pl.ds(base + k * CHUNK, CHUNK)])
        rem = ROWS_PER_TILE % CHUNK
        if rem:
            pltpu.sync_copy(rows0.at[pl.ds(0, rem)],
                            acc.at[pl.ds(base + ROWS_PER_TILE - rem, rem)])
        return base

    def make_part_loop(T, isrc, idst, acc, isv, idv, bufs, sems, extra):
        def gather(j, buf, sem):
            pltpu.async_copy(T.at[isv.at[j]], buf, sem)

        def gwait(j, buf, sem):
            pltpu.make_async_copy(T.at[isv.at[j]], buf, sem).wait()

        def scatter(j, buf):
            pltpu.sync_copy(buf, acc.at[idv.at[j]], add=True)
            if extra is not None:
                dh, bh, ones_v = extra
                pltpu.sync_copy(ones_v, dh.at[isv.at[j]], add=True)
                pltpu.sync_copy(ones_v, bh.at[idv.at[j]], add=True)

        nbuf = len(bufs)

        def ring_step(i, carry):
            j0 = nbuf * i
            for b in range(nbuf):
                gwait(j0 + b, bufs[b], sems[b])
                scatter(j0 + b, bufs[b])

                @pl.when(i < PART // nbuf - 1)
                def _next(b=b):
                    gather(j0 + nbuf + b, bufs[b], sems[b])

            return carry

        def run(start, nch):
            for p in range(nch // PART):
                pltpu.sync_copy(isrc.at[pl.ds(start + p * PART, PART)], isv)
                pltpu.sync_copy(idst.at[pl.ds(start + p * PART, PART)], idv)
                for b in range(nbuf):
                    gather(b, bufs[b], sems[b])
                lax.fori_loop(0, PART // nbuf, ring_step, 0)

        return run

    def core_dispatch(run, c, s):
        @pl.when(c == 0)
        def _c0():
            run(s * nch0, nch0)

        @pl.when(c == 1)
        def _c1():
            run(NS * nch0 + s * nch1, nch1)

    if with_counts:
        def body(T, isrc, idst, P, Dh, Bh,
                 acc, isv, idv, rows0, rows1, rows2, rows3,
                 sem0, sem1, sem2, sem3,
                 dh, bh, ones_v, zline):
            c = lax.axis_index("c")
            s = lax.axis_index("s")
            base = prologue(acc, rows0, s)
            _fill_1d(ones_v, CHUNK // 16, 1.0)
            _fill_1d(zline, 40, 0.0)
            pltpu.sync_copy(zline.at[pl.ds(0, ROWS_PER_TILE)],
                            dh.at[pl.ds(base, ROWS_PER_TILE)])
            pltpu.sync_copy(zline.at[pl.ds(0, ROWS_PER_TILE)],
                            bh.at[pl.ds(base, ROWS_PER_TILE)])
            plsc.subcore_barrier()
            run = make_part_loop(T, isrc, idst, acc, isv, idv,
                                 (rows0, rows1, rows2, rows3),
                                 (sem0, sem1, sem2, sem3), (dh, bh, ones_v))
            core_dispatch(run, c, s)
            plsc.subcore_barrier()
            pltpu.sync_copy(acc.at[pl.ds(base, ROWS_PER_TILE)],
                            P.at[c, pl.ds(base, ROWS_PER_TILE)])

            @pl.when(s == 0)
            def _drain_hists():
                pltpu.sync_copy(dh, Dh.at[pl.ds(c * NPAD, NPAD)])
                pltpu.sync_copy(bh, Bh.at[pl.ds(c * NPAD, NPAD)])
    else:
        def body(T, isrc, idst, P,
                 acc, isv, idv, rows0, rows1, rows2, rows3,
                 sem0, sem1, sem2, sem3):
            c = lax.axis_index("c")
            s = lax.axis_index("s")
            base = prologue(acc, rows0, s)
            plsc.subcore_barrier()
            run = make_part_loop(T, isrc, idst, acc, isv, idv,
                                 (rows0, rows1, rows2, rows3),
                                 (sem0, sem1, sem2, sem3), None)
            core_dispatch(run, c, s)
            plsc.subcore_barrier()
            pltpu.sync_copy(acc.at[pl.ds(base, ROWS_PER_TILE)],
                            P.at[c, pl.ds(base, ROWS_PER_TILE)])

    return pl.kernel(body, out_type=tuple(out_type), mesh=mesh,
                     scratch_types=tuple(scratch))


def _tc_matmul_body(x_ref, w_ref, o_ref):
    o_ref[...] = jnp.dot(x_ref[...], w_ref[...], preferred_element_type=F32)


def _tc_inv_body(npad_e, dh_ref, bh_ref, dinv_ref, binv_ref):
    row = lax.broadcasted_iota(jnp.int32, (NPAD // C, C), 0)
    col = lax.broadcasted_iota(jnp.int32, (NPAD // C, C), 1)
    corr = jnp.where((row == 0) & (col == 0), float(npad_e), 0.0)
    d = dh_ref[0] + dh_ref[1] - corr
    b = bh_ref[0] + bh_ref[1]
    dinv_ref[...] = jnp.where(d > 0, 1.0 / d, 0.0)
    binv_ref[...] = jnp.where(b > 0, 1.0 / b, 0.0)


def _tc_scale_body(p_ref, inv_ref, o_ref):
    o_ref[...] = (p_ref[0] + p_ref[1]) * inv_ref[...]


def _tc_layer_end_body(p_ref, inv_ref, b_ref, w_ref, o_ref):
    h = jax.nn.relu((p_ref[0] + p_ref[1]) * inv_ref[...] + b_ref[...])
    o_ref[...] = jnp.dot(h, w_ref[...], preferred_element_type=F32)


def _tc_final_body(p_ref, inv_ref, b_ref, batch_ref, wfc_ref, bfc_ref, o_ref):
    h = jax.nn.relu((p_ref[0] + p_ref[1]) * inv_ref[...] + b_ref[...])
    gids = lax.broadcasted_iota(jnp.int32, (N_GRAPHS, 1), 0)
    oht = (gids == batch_ref[...]).astype(F32)
    sums = jnp.dot(oht, h, preferred_element_type=F32,
                   precision=lax.Precision.HIGHEST)
    cnts = jnp.dot(oht, jnp.ones((NPAD, 1), F32),
                   preferred_element_type=F32, precision=lax.Precision.HIGHEST)
    pooled = sums / jnp.maximum(cnts, 1.0)
    o_ref[...] = jnp.dot(pooled, wfc_ref[...],
                         preferred_element_type=F32) + bfc_ref[...]


def _tc(body, out_shape, *args):
    return pl.pallas_call(body, out_shape=out_shape)(*args)


def kernel(x, edge_index, batch, W1, b1, W2, b2, Wfc, bfc):
    nnz = edge_index.shape[1]
    nch0, nch1 = _chunk_split(nnz)
    total_chunks = NS * (nch0 + nch1)
    nnz_pad = total_chunks * CHUNK
    npad_e = nnz_pad - nnz

    node = edge_index[0]
    edge = edge_index[1]
    pad0 = jnp.zeros((npad_e,), jnp.int32)
    padT = jnp.full((npad_e,), TRASH, jnp.int32)
    shape2 = (total_chunks, CHUNK)
    node0 = jnp.concatenate([node, pad0]).reshape(shape2)
    nodeT = jnp.concatenate([node, padT]).reshape(shape2)
    edge0 = jnp.concatenate([edge, pad0]).reshape(shape2)
    edgeT = jnp.concatenate([edge, padT]).reshape(shape2)

    spmm_c = _make_spmm(N_NODES, nch0, nch1, True)
    spmm_n = _make_spmm(NPAD, nch0, nch1, False)

    t1 = _tc(_tc_matmul_body, jax.ShapeDtypeStruct((N_NODES, C), F32), x, W1)
    P, Dh, Bh = spmm_c(t1, node0, edgeT)
    dinv, binv = _tc(
        functools.partial(_tc_inv_body, npad_e),
        (jax.ShapeDtypeStruct((NPAD // C, C), F32),) * 2,
        Dh.reshape(NC, NPAD // C, C), Bh.reshape(NC, NPAD // C, C))
    dinv_col = dinv.reshape(NPAD, 1)
    binv_col = binv.reshape(NPAD, 1)
    m1 = _tc(_tc_scale_body, jax.ShapeDtypeStruct((NPAD, C), F32), P, binv_col)
    (P,) = spmm_n(m1, edge0, nodeT)
    t2 = _tc(_tc_layer_end_body, jax.ShapeDtypeStruct((NPAD, C), F32),
             P, dinv_col, b1.reshape(1, C), W2)

    (P,) = spmm_n(t2, node0, edgeT)
    m2 = _tc(_tc_scale_body, jax.ShapeDtypeStruct((NPAD, C), F32), P, binv_col)
    (P,) = spmm_n(m2, edge0, nodeT)

    batch_row = jnp.concatenate(
        [batch, jnp.full((NPAD - N_NODES,), N_GRAPHS, jnp.int32)]).reshape(1, NPAD)
    out = _tc(_tc_final_body, jax.ShapeDtypeStruct((N_GRAPHS, 1), F32),
              P, dinv_col, b2.reshape(1, C), batch_row, Wfc, bfc.reshape(1, 1))
    return out.reshape(-1)
